# baseline (speedup 1.0000x reference)
import jax
import jax.numpy as jnp
from jax.experimental import pallas as pl
from jax.experimental.pallas import tpu as pltpu

N = 10000
F = 128
H = 128
C = 64

BI = 2048
BK = 2048
NI = 5
NK = 5
NPAD = BI * NI
BI2 = 1024

_BF = jnp.bfloat16
_F8 = jnp.float8_e4m3fn


def _mm(a, b):
    return jnp.dot(a, b, preferred_element_type=jnp.float32)


def _mm_tn(lhs, rhs):
    return jax.lax.dot_general(lhs, rhs, (((0,), (1,)), ((), ())),
                               preferred_element_type=jnp.float32)


def _mm_nt(lhs, rhs):
    return jax.lax.dot_general(lhs, rhs, (((1,), (1,)), ((), ())),
                               preferred_element_type=jnp.float32)


def _pass1_kernel(adj_ref, x_ref, w1_ref, h_ref, q_ref, acc_ref):
    i = pl.program_id(0)
    k = pl.program_id(1)
    xk = x_ref[pl.ds(pl.multiple_of(k * BK, 8), BK), :]

    @pl.when(k < NK - 1)
    def _():
        a = adj_ref[...]
        q_ref[...] = (a - 0.5).astype(_F8)
        part = _mm(_mm(a, xk), w1_ref[F:2 * F, :])

        @pl.when(k == 0)
        def _():
            acc_ref[...] = part

        @pl.when(k > 0)
        def _():
            acc_ref[...] += part

    @pl.when(k == NK - 1)
    def _():
        col = jax.lax.broadcasted_iota(jnp.int32, (BI, BK), 1)
        a = jnp.where(col < N - (NK - 1) * BK, adj_ref[...], 0.0)
        q_ref[...] = (a - 0.5).astype(_F8)
        part = _mm(_mm(a, xk), w1_ref[F:2 * F, :])
        xi = x_ref[pl.ds(pl.multiple_of(i * BI, 8), BI), :]
        h = _mm(xi, w1_ref[0:F, :]) + acc_ref[...] + part
        h = jnp.maximum(h, 0.0)

        @pl.when(i == NI - 1)
        def _():
            row = jax.lax.broadcasted_iota(jnp.int32, (BI, F), 0)
            h_ref[...] = jnp.where(row < N - (NI - 1) * BI, h, 0.0).astype(_BF)

        @pl.when(i < NI - 1)
        def _():
            h_ref[...] = h.astype(_BF)


def _pass2_kernel(q_ref, h_ref, w2t_ref, wlin_ref, corr_ref, out_ref):
    i = pl.program_id(0)
    s2t = _mm_tn(h_ref[...], q_ref[...])
    hi = h_ref[pl.ds(pl.multiple_of(i * BI2, 8), BI2), :].astype(jnp.float32)
    h2t = jnp.maximum(_mm_nt(w2t_ref[:, 0:H], hi)
                      + _mm(w2t_ref[:, H:2 * H], s2t)
                      + corr_ref[:, 0:1], 0.0)
    yt = _mm(wlin_ref[...], h2t)
    m = jnp.max(yt, axis=0, keepdims=True)
    e = jnp.exp(yt - m)
    s = jnp.sum(e, axis=0, keepdims=True)
    out_ref[...] = yt - m - jnp.log(s)


def kernel(x, adj, W1, W2, Wlin):
    xp = jnp.zeros((NPAD, F), jnp.float32).at[:N, :].set(x)
    w2st = W2.T

    grid = (NI, NK)
    adj_spec = pl.BlockSpec((BI, BK), lambda i, k: (i, k))
    full_spec = pl.BlockSpec((NPAD, F), lambda i, k: (0, 0))
    w_spec = pl.BlockSpec((2 * F, H), lambda i, k: (0, 0))
    params = pltpu.CompilerParams(
        dimension_semantics=("parallel", "arbitrary"))

    hp, q8 = pl.pallas_call(
        _pass1_kernel,
        grid=grid,
        in_specs=[adj_spec, full_spec, w_spec],
        out_specs=[pl.BlockSpec((BI, F), lambda i, k: (i, 0)), adj_spec],
        out_shape=[jax.ShapeDtypeStruct((NPAD, F), _BF),
                   jax.ShapeDtypeStruct((NPAD, NPAD), _F8)],
        scratch_shapes=[pltpu.VMEM((BI, F), jnp.float32)],
        compiler_params=params,
    )(adj, xp, W1)

    ch = jnp.sum(hp.astype(jnp.float32), axis=0)
    corr = jnp.broadcast_to((0.5 * (ch @ W2[H:, :]))[:, None], (H, 128))

    out_t = pl.pallas_call(
        _pass2_kernel,
        grid=(NPAD // BI2,),
        in_specs=[pl.BlockSpec((BI2, NPAD), lambda i: (i, 0)),
                  pl.BlockSpec((NPAD, F), lambda i: (0, 0)),
                  pl.BlockSpec((H, 2 * H), lambda i: (0, 0)),
                  pl.BlockSpec((C, H), lambda i: (0, 0)),
                  pl.BlockSpec((H, 128), lambda i: (0, 0))],
        out_specs=pl.BlockSpec((C, BI2), lambda i: (0, i)),
        out_shape=jax.ShapeDtypeStruct((C, N), jnp.float32),
        compiler_params=pltpu.CompilerParams(
            dimension_semantics=("parallel",)),
    )(q8, hp, w2st, Wlin, corr)

    return out_t.T


# speedup vs baseline: 1.0038x; 1.0038x over previous
import jax
import jax.numpy as jnp
from jax.experimental import pallas as pl
from jax.experimental.pallas import tpu as pltpu

N = 10000
F = 128
H = 128
C = 64

BI = 2048
BK = 2048
NI = 5
NK = 5
NPAD = BI * NI
BI2 = 2048

_BF = jnp.bfloat16
_F8 = jnp.float8_e4m3fn


def _mm(a, b):
    return jnp.dot(a, b, preferred_element_type=jnp.float32)


def _mm_tn(lhs, rhs):
    return jax.lax.dot_general(lhs, rhs, (((0,), (1,)), ((), ())),
                               preferred_element_type=jnp.float32)


def _mm_nt(lhs, rhs):
    return jax.lax.dot_general(lhs, rhs, (((1,), (1,)), ((), ())),
                               preferred_element_type=jnp.float32)


def _pass1_kernel(adj_ref, x_ref, w1_ref, h_ref, q_ref, acc_ref):
    i = pl.program_id(0)
    k = pl.program_id(1)
    xk = x_ref[pl.ds(pl.multiple_of(k * BK, 8), BK), :]

    @pl.when(k < NK - 1)
    def _():
        a = adj_ref[...]
        q_ref[...] = (a - 0.5).astype(_F8)
        part = _mm(_mm(a, xk), w1_ref[F:2 * F, :])

        @pl.when(k == 0)
        def _():
            acc_ref[...] = part

        @pl.when(k > 0)
        def _():
            acc_ref[...] += part

    @pl.when(k == NK - 1)
    def _():
        col = jax.lax.broadcasted_iota(jnp.int32, (BI, BK), 1)
        a = jnp.where(col < N - (NK - 1) * BK, adj_ref[...], 0.0)
        q_ref[...] = (a - 0.5).astype(_F8)
        part = _mm(_mm(a, xk), w1_ref[F:2 * F, :])
        xi = x_ref[pl.ds(pl.multiple_of(i * BI, 8), BI), :]
        h = _mm(xi, w1_ref[0:F, :]) + acc_ref[...] + part
        h = jnp.maximum(h, 0.0)

        @pl.when(i == NI - 1)
        def _():
            row = jax.lax.broadcasted_iota(jnp.int32, (BI, F), 0)
            h_ref[...] = jnp.where(row < N - (NI - 1) * BI, h, 0.0).astype(_BF)

        @pl.when(i < NI - 1)
        def _():
            h_ref[...] = h.astype(_BF)


def _pass2_kernel(q_ref, h_ref, w2t_ref, wlin_ref, corr_ref, out_ref):
    i = pl.program_id(0)
    s2t = _mm_tn(h_ref[...], q_ref[...])
    hi = h_ref[pl.ds(pl.multiple_of(i * BI2, 8), BI2), :].astype(jnp.float32)
    h2t = jnp.maximum(_mm_nt(w2t_ref[:, 0:H], hi)
                      + _mm(w2t_ref[:, H:2 * H], s2t)
                      + corr_ref[:, 0:1], 0.0)
    yt = _mm(wlin_ref[...], h2t)
    m = jnp.max(yt, axis=0, keepdims=True)
    e = jnp.exp(yt - m)
    s = jnp.sum(e, axis=0, keepdims=True)
    out_ref[...] = yt - m - jnp.log(s)


def kernel(x, adj, W1, W2, Wlin):
    xp = jnp.zeros((NPAD, F), jnp.float32).at[:N, :].set(x)
    w2st = W2.T

    grid = (NI, NK)
    adj_spec = pl.BlockSpec((BI, BK), lambda i, k: (i, k))
    full_spec = pl.BlockSpec((NPAD, F), lambda i, k: (0, 0))
    w_spec = pl.BlockSpec((2 * F, H), lambda i, k: (0, 0))
    params = pltpu.CompilerParams(
        dimension_semantics=("parallel", "arbitrary"))

    hp, q8 = pl.pallas_call(
        _pass1_kernel,
        grid=grid,
        in_specs=[adj_spec, full_spec, w_spec],
        out_specs=[pl.BlockSpec((BI, F), lambda i, k: (i, 0)), adj_spec],
        out_shape=[jax.ShapeDtypeStruct((NPAD, F), _BF),
                   jax.ShapeDtypeStruct((NPAD, NPAD), _F8)],
        scratch_shapes=[pltpu.VMEM((BI, F), jnp.float32)],
        compiler_params=params,
    )(adj, xp, W1)

    ch = jnp.sum(hp.astype(jnp.float32), axis=0)
    corr = jnp.broadcast_to((0.5 * (ch @ W2[H:, :]))[:, None], (H, 128))

    out_t = pl.pallas_call(
        _pass2_kernel,
        grid=(NPAD // BI2,),
        in_specs=[pl.BlockSpec((BI2, NPAD), lambda i: (i, 0)),
                  pl.BlockSpec((NPAD, F), lambda i: (0, 0)),
                  pl.BlockSpec((H, 2 * H), lambda i: (0, 0)),
                  pl.BlockSpec((C, H), lambda i: (0, 0)),
                  pl.BlockSpec((H, 128), lambda i: (0, 0))],
        out_specs=pl.BlockSpec((C, BI2), lambda i: (0, i)),
        out_shape=jax.ShapeDtypeStruct((C, N), jnp.float32),
        compiler_params=pltpu.CompilerParams(
            dimension_semantics=("parallel",)),
    )(q8, hp, w2st, Wlin, corr)

    return out_t.T
